# baseline (speedup 1.0000x reference)
import jax
import jax.numpy as jnp
from jax import lax
from jax.experimental import pallas as pl
from jax.experimental.pallas import tpu as pltpu

_IM_DIMX = 1333
_IM_DIMY = 800

_N_STREAMS = 8
_CHUNK = 2


def _make_body(b, h, w, L, N, n_j, n_streams):
    sx = float(w) / _IM_DIMX
    sy = float(h) / _IM_DIMY
    tn = float(b * h * w)

    def _body(label_ref, *refs):
        x_refs = refs[:n_streams]
        out_ref = refs[n_streams]
        pi_ref = refs[n_streams + 1]
        acc_ref = refs[n_streams + 2]
        i = pl.program_id(0)
        j = pl.program_id(1)

        @pl.when((i == 0) & (j == 0))
        def _rasterize():
            lbl = label_ref[...].astype(jnp.float32)
            x1 = jnp.clip(jnp.round(lbl[:, 0] * sx), 0.0, float(w - 1))
            y1 = jnp.clip(jnp.round(lbl[:, 1] * sy), 0.0, float(h - 1))
            x2 = jnp.clip(jnp.round(lbl[:, 2] * sx), 0.0, float(w))
            y2 = jnp.clip(jnp.round(lbl[:, 3] * sy), 0.0, float(h))
            valid = jnp.logical_not(
                (x2 <= x1) | (y2 <= y1) | (x1 + x2 >= float(w)) | (y1 + y2 >= float(h))
            )
            vf = valid.astype(jnp.float32)
            x1i = x1.astype(jnp.int32)
            y1i = y1.astype(jnp.int32)
            x2i = x2.astype(jnp.int32)
            y2i = y2.astype(jnp.int32)
            rows = lax.broadcasted_iota(jnp.int32, (h, N), 0)
            rt = ((rows >= y1i[None, :]) & (rows < y2i[None, :])).astype(jnp.float32)
            rt = rt * vf[None, :]
            cols = lax.broadcasted_iota(jnp.int32, (N, w), 1)
            cm = ((cols >= x1i[:, None]) & (cols < x2i[:, None])).astype(jnp.float32)
            count = lax.dot_general(
                rt, cm, (((1,), (0,)), ((), ())), preferred_element_type=jnp.float32
            )
            covered = jnp.sum((count > 0.5).astype(jnp.float32))
            pi_ref[0] = covered / tn

        s = jnp.sum(x_refs[0][...])
        for r in x_refs[1:]:
            s = s + jnp.sum(r[...])

        @pl.when(j == 0)
        def _init():
            acc_ref[i] = s

        @pl.when(j != 0)
        def _accum():
            acc_ref[i] = acc_ref[i] + s

        @pl.when((i == L - 1) & (j == n_j - 1))
        def _combine():
            pi = pi_ref[0]
            tot = 0.0
            for k in range(L):
                tot = tot + (acc_ref[k] / tn - pi) ** 2
            out_ref[0, 0] = tot / float(L)

    return _body


def _stream_spec(chunk, n_streams, C, H, W, s):
    return pl.BlockSpec(
        (1, chunk, H, W), lambda i, j: (i, j * n_streams + s, 0, 0)
    )


def kernel(h_masks, label):
    L, B, C, H, W = h_masks.shape
    K, Nb, _ = label.shape
    N = K * Nb
    ns = _N_STREAMS
    ck = _CHUNK
    flat = jnp.reshape(h_masks, (L, B * C, H, W))
    n_j = (B * C) // (ns * ck)

    boxes = jnp.reshape(label, (N, 4))
    specs = [pl.BlockSpec(memory_space=pltpu.VMEM)]
    for s in range(ns):
        specs.append(_stream_spec(ck, ns, C, H, W, s))

    out = pl.pallas_call(
        _make_body(B, H, W, L, N, n_j, ns),
        grid=(L, n_j),
        in_specs=specs,
        out_specs=pl.BlockSpec(memory_space=pltpu.SMEM),
        out_shape=jax.ShapeDtypeStruct((1, 1), jnp.float32),
        scratch_shapes=[
            pltpu.SMEM((1,), jnp.float32),
            pltpu.SMEM((L,), jnp.float32),
        ],
    )(boxes, *([flat] * ns))
    return out[0, 0]


# speedup vs baseline: 1.0805x; 1.0805x over previous
import jax
import jax.numpy as jnp
from jax import lax
from jax.experimental import pallas as pl
from jax.experimental.pallas import tpu as pltpu

_IM_DIMX = 1333
_IM_DIMY = 800

_N_STREAMS = 4
_CHUNK = 4


def _make_body(b, h, w, L, N, n_j, n_streams):
    sx = float(w) / _IM_DIMX
    sy = float(h) / _IM_DIMY
    tn = float(b * h * w)

    def _body(label_ref, *refs):
        x_refs = refs[:n_streams]
        out_ref = refs[n_streams]
        pi_ref = refs[n_streams + 1]
        acc_ref = refs[n_streams + 2]
        i = pl.program_id(0)
        j = pl.program_id(1)

        @pl.when((i == 0) & (j == 0))
        def _rasterize():
            lbl = label_ref[...].astype(jnp.float32)
            x1 = jnp.clip(jnp.round(lbl[:, 0] * sx), 0.0, float(w - 1))
            y1 = jnp.clip(jnp.round(lbl[:, 1] * sy), 0.0, float(h - 1))
            x2 = jnp.clip(jnp.round(lbl[:, 2] * sx), 0.0, float(w))
            y2 = jnp.clip(jnp.round(lbl[:, 3] * sy), 0.0, float(h))
            valid = jnp.logical_not(
                (x2 <= x1) | (y2 <= y1) | (x1 + x2 >= float(w)) | (y1 + y2 >= float(h))
            )
            vf = valid.astype(jnp.float32)
            x1i = x1.astype(jnp.int32)
            y1i = y1.astype(jnp.int32)
            x2i = x2.astype(jnp.int32)
            y2i = y2.astype(jnp.int32)
            rows = lax.broadcasted_iota(jnp.int32, (h, N), 0)
            rt = ((rows >= y1i[None, :]) & (rows < y2i[None, :])).astype(jnp.float32)
            rt = rt * vf[None, :]
            cols = lax.broadcasted_iota(jnp.int32, (N, w), 1)
            cm = ((cols >= x1i[:, None]) & (cols < x2i[:, None])).astype(jnp.float32)
            count = lax.dot_general(
                rt, cm, (((1,), (0,)), ((), ())), preferred_element_type=jnp.float32
            )
            covered = jnp.sum((count > 0.5).astype(jnp.float32))
            pi_ref[0] = covered / tn

        s = jnp.sum(x_refs[0][...])
        for r in x_refs[1:]:
            s = s + jnp.sum(r[...])

        @pl.when(j == 0)
        def _init():
            acc_ref[i] = s

        @pl.when(j != 0)
        def _accum():
            acc_ref[i] = acc_ref[i] + s

        @pl.when((i == L - 1) & (j == n_j - 1))
        def _combine():
            pi = pi_ref[0]
            tot = 0.0
            for k in range(L):
                tot = tot + (acc_ref[k] / tn - pi) ** 2
            out_ref[0, 0] = tot / float(L)

    return _body


def _stream_spec(chunk, n_streams, C, H, W, s):
    return pl.BlockSpec(
        (1, chunk, H, W), lambda i, j: (i, j * n_streams + s, 0, 0)
    )


def kernel(h_masks, label):
    L, B, C, H, W = h_masks.shape
    K, Nb, _ = label.shape
    N = K * Nb
    ns = _N_STREAMS
    ck = _CHUNK
    flat = jnp.reshape(h_masks, (L, B * C, H, W))
    n_j = (B * C) // (ns * ck)

    boxes = jnp.reshape(label, (N, 4))
    specs = [pl.BlockSpec(memory_space=pltpu.VMEM)]
    for s in range(ns):
        specs.append(_stream_spec(ck, ns, C, H, W, s))

    out = pl.pallas_call(
        _make_body(B, H, W, L, N, n_j, ns),
        grid=(L, n_j),
        in_specs=specs,
        out_specs=pl.BlockSpec(memory_space=pltpu.SMEM),
        out_shape=jax.ShapeDtypeStruct((1, 1), jnp.float32),
        scratch_shapes=[
            pltpu.SMEM((1,), jnp.float32),
            pltpu.SMEM((L,), jnp.float32),
        ],
    )(boxes, *([flat] * ns))
    return out[0, 0]
